# baseline (speedup 1.0000x reference)
import functools

import jax
import jax.numpy as jnp
from jax import lax
from jax.experimental import pallas as pl
from jax.experimental.pallas import tpu as pltpu
from jax.experimental.pallas import tpu_sc as plsc

N = 10000
E = 320000
D = 128
G = 512
NP = 10240
GP = 768
NC = 2
NS = 16
TILES = NC * NS
ROWS_PER_TILE = NP // TILES
ROWS_PER_SUB = NP // NS

ECHUNK = 64
NCHUNK = 160
HCHUNK = 40
NBUF = 4
EP = TILES * NCHUNK * ECHUNK

PEC = 128
PNCH = 80

_mesh = plsc.VectorSubcoreMesh(core_axis_name="c", subcore_axis_name="s")


def _wid():
    return lax.axis_index("s") * NC + lax.axis_index("c")


@functools.partial(
    pl.kernel,
    out_type=(
        jax.ShapeDtypeStruct((NP, D), jnp.float32),
        jax.ShapeDtypeStruct((NP,), jnp.float32),
        jax.ShapeDtypeStruct((NP,), jnp.float32),
    ),
    mesh=_mesh,
    scratch_types=[
        pltpu.VMEM((PNCH, PEC), jnp.int32),
        pltpu.VMEM((PEC,), jnp.float32),
        pltpu.VMEM((64,), jnp.int32),
        pltpu.VMEM((64, D), jnp.float32),
        pltpu.VMEM((64,), jnp.float32),
        pltpu.VMEM((ROWS_PER_SUB,), jnp.float32),
        pltpu.VMEM_SHARED((NP,), jnp.float32),
        pltpu.SemaphoreType.DMA,
    ],
)
def _sc_prep(srcp, xp, emb, z1h, onesh, h0_out, deg0_out, deg1_out,
             sidx, onesv, xidx, rows, z1v, dbuf, acc1, sem):
    c = lax.axis_index("c")
    s = lax.axis_index("s")
    wid = _wid()
    pltpu.sync_copy(z1h, z1v)
    pltpu.sync_copy(onesh, onesv)

    def zero_body(j, _):
        pltpu.sync_copy(z1v, acc1.at[pl.ds(s * ROWS_PER_SUB + j * 64, 64)])
        return _
    lax.fori_loop(0, ROWS_PER_SUB // 64, zero_body, None)

    def emb_body(j, _):
        base = wid * ROWS_PER_TILE + j * 64
        pltpu.sync_copy(xp.at[pl.ds(base, 64)], xidx)
        pltpu.async_copy(emb.at[xidx], rows, sem).wait()
        pltpu.sync_copy(rows, h0_out.at[pl.ds(base, 64)])
        return _
    lax.fori_loop(0, ROWS_PER_TILE // 64, emb_body, None)

    pltpu.sync_copy(srcp.at[wid], sidx)
    plsc.subcore_barrier()

    def deg_body(i, _):
        pltpu.sync_copy(onesv, acc1.at[sidx.at[i]], add=True)
        return _
    lax.fori_loop(0, PNCH, deg_body, None)

    plsc.subcore_barrier()
    sl = pl.ds(s * ROWS_PER_SUB, ROWS_PER_SUB)
    pltpu.sync_copy(acc1.at[sl], dbuf)

    @pl.when(c == 0)
    def _w0():
        pltpu.sync_copy(dbuf, deg0_out.at[sl])

    @pl.when(c == 1)
    def _w1():
        pltpu.sync_copy(dbuf, deg1_out.at[sl])


@functools.partial(
    pl.kernel,
    out_type=jax.ShapeDtypeStruct((NC, NP, D), jnp.float32),
    mesh=_mesh,
    scratch_types=[
        pltpu.VMEM((HCHUNK, ECHUNK), jnp.int32),
        pltpu.VMEM((HCHUNK, ECHUNK), jnp.int32),
        pltpu.VMEM((ECHUNK, D), jnp.float32),
        pltpu.VMEM((ECHUNK, D), jnp.float32),
        pltpu.VMEM((ECHUNK, D), jnp.float32),
        pltpu.VMEM((ECHUNK, D), jnp.float32),
        pltpu.VMEM_SHARED((NP, D), jnp.float32),
        pltpu.SemaphoreType.DMA,
        pltpu.SemaphoreType.DMA,
        pltpu.SemaphoreType.DMA,
        pltpu.SemaphoreType.DMA,
        pltpu.SemaphoreType.DMA,
        pltpu.SemaphoreType.DMA,
        pltpu.SemaphoreType.DMA,
        pltpu.SemaphoreType.DMA,
    ],
)
def _sc_scatter(hn, src2d, dst2d, zh, agg_out,
                sidx, didx, r0, r1, r2, r3, acc,
                g0, g1, g2, g3, t0, t1, t2, t3):
    c = lax.axis_index("c")
    s = lax.axis_index("s")
    wid = _wid()
    rows = (r0, r1, r2, r3)
    gsem = (g0, g1, g2, g3)
    tsem = (t0, t1, t2, t3)
    sl = pl.ds(s * ROWS_PER_SUB, ROWS_PER_SUB)

    pltpu.sync_copy(zh, acc.at[sl])
    plsc.subcore_barrier()

    def _g_start(k, b):
        pltpu.async_copy(hn.at[sidx.at[k]], rows[b], gsem[b])

    def _g_wait(k, b):
        pltpu.make_async_copy(hn.at[sidx.at[k]], rows[b], gsem[b]).wait()

    def _t_start(k, b):
        pltpu.async_copy(rows[b], acc.at[didx.at[k]], tsem[b], add=True)

    def _t_wait(k, b):
        pltpu.make_async_copy(rows[b], acc.at[didx.at[k]], tsem[b]).wait()

    for h in range(NCHUNK // HCHUNK):
        pltpu.sync_copy(src2d.at[wid, pl.ds(h * HCHUNK, HCHUNK)], sidx)
        pltpu.sync_copy(dst2d.at[wid, pl.ds(h * HCHUNK, HCHUNK)], didx)
        for b in range(NBUF):
            _g_start(b, b)

        def group(j, _):
            for b in range(NBUF):
                k = j * NBUF + b
                _g_wait(k, b)
                _t_start(k, b)
                b2 = (b + 2) % NBUF

                @pl.when(k >= 2)
                def _retire():
                    _t_wait(k - 2, b2)

                    @pl.when(k + 2 < HCHUNK)
                    def _refill():
                        _g_start(k + 2, b2)
            return _
        lax.fori_loop(0, HCHUNK // NBUF, group, None)

        _t_wait(HCHUNK - 2, (HCHUNK - 2) % NBUF)
        _t_wait(HCHUNK - 1, (HCHUNK - 1) % NBUF)

    plsc.subcore_barrier()
    pltpu.sync_copy(acc.at[sl], agg_out.at[c, sl])


_GROWS = GP // NS


@functools.partial(
    pl.kernel,
    out_type=(
        jax.ShapeDtypeStruct((NC, GP, D), jnp.float32),
        jax.ShapeDtypeStruct((GP,), jnp.float32),
        jax.ShapeDtypeStruct((GP,), jnp.float32),
    ),
    mesh=_mesh,
    scratch_types=[
        pltpu.VMEM((ROWS_PER_TILE // 64, 64), jnp.int32),
        pltpu.VMEM((64, D), jnp.float32),
        pltpu.VMEM((64,), jnp.float32),
        pltpu.VMEM((64, D), jnp.float32),
        pltpu.VMEM((_GROWS,), jnp.float32),
        pltpu.VMEM_SHARED((GP, D), jnp.float32),
        pltpu.VMEM_SHARED((GP,), jnp.float32),
    ],
)
def _sc_pool(h4, ptr2d, z2h, z1h, onesh, pooled_out, cnt0_out, cnt1_out,
             pidx, rowb, onesv, zb, z1v, acc_r, acc_c):
    c = lax.axis_index("c")
    s = lax.axis_index("s")
    wid = _wid()
    pltpu.sync_copy(z2h.at[pl.ds(0, 64)], zb)
    pltpu.sync_copy(z1h.at[pl.ds(0, _GROWS)], z1v)
    pltpu.sync_copy(onesh.at[pl.ds(0, 64)], onesv)
    sl = pl.ds(s * _GROWS, _GROWS)
    pltpu.sync_copy(zb.at[pl.ds(0, _GROWS)], acc_r.at[sl])
    pltpu.sync_copy(z1v, acc_c.at[sl])
    nch = ROWS_PER_TILE // 64
    pltpu.sync_copy(ptr2d.at[wid], pidx)
    plsc.subcore_barrier()

    def body(j, _):
        pltpu.sync_copy(h4.at[pl.ds(wid * ROWS_PER_TILE + j * 64, 64)], rowb)
        pltpu.sync_copy(rowb, acc_r.at[pidx.at[j]], add=True)
        pltpu.sync_copy(onesv, acc_c.at[pidx.at[j]], add=True)
        return _
    lax.fori_loop(0, nch, body, None)

    plsc.subcore_barrier()
    pltpu.sync_copy(acc_r.at[sl], zb.at[pl.ds(0, _GROWS)])
    pltpu.sync_copy(zb.at[pl.ds(0, _GROWS)], pooled_out.at[c, sl])
    pltpu.sync_copy(acc_c.at[sl], z1v)

    @pl.when(c == 0)
    def _w0():
        pltpu.sync_copy(z1v, cnt0_out.at[sl])

    @pl.when(c == 1)
    def _w1():
        pltpu.sync_copy(z1v, cnt1_out.at[sl])


_RB = 512
_NBLK = NP // _RB


def _tc_prep_body(h0_ref, deg0_ref, deg1_ref, hn_ref, rdeg_ref):
    dg = deg0_ref[...] + deg1_ref[...]
    r = lax.rsqrt(dg + 1.0)
    rdeg_ref[...] = r
    hn_ref[...] = h0_ref[...] * r


def _tc_prep(h0, deg0, deg1):
    return pl.pallas_call(
        _tc_prep_body,
        grid=(_NBLK,),
        in_specs=[
            pl.BlockSpec((_RB, D), lambda i: (i, 0)),
            pl.BlockSpec((_RB, 1), lambda i: (i, 0)),
            pl.BlockSpec((_RB, 1), lambda i: (i, 0)),
        ],
        out_specs=[
            pl.BlockSpec((_RB, D), lambda i: (i, 0)),
            pl.BlockSpec((_RB, 1), lambda i: (i, 0)),
        ],
        out_shape=[
            jax.ShapeDtypeStruct((NP, D), jnp.float32),
            jax.ShapeDtypeStruct((NP, 1), jnp.float32),
        ],
    )(h0, deg0, deg1)


def _tc_layer_body(agg_ref, hn_ref, w_ref, b_ref, sc_ref, out_ref):
    a = agg_ref[0] + agg_ref[1] + hn_ref[...]
    y = jnp.dot(a, w_ref[...], preferred_element_type=jnp.float32)
    y = jnp.maximum(y + b_ref[...], 0.0)
    out_ref[...] = y * sc_ref[...]


def _tc_layer(agg2, hn, w, b, scale):
    return pl.pallas_call(
        _tc_layer_body,
        grid=(_NBLK,),
        in_specs=[
            pl.BlockSpec((NC, _RB, D), lambda i: (0, i, 0)),
            pl.BlockSpec((_RB, D), lambda i: (i, 0)),
            pl.BlockSpec((D, D), lambda i: (0, 0)),
            pl.BlockSpec((1, D), lambda i: (0, 0)),
            pl.BlockSpec((_RB, 1), lambda i: (i, 0)),
        ],
        out_specs=pl.BlockSpec((_RB, D), lambda i: (i, 0)),
        out_shape=jax.ShapeDtypeStruct((NP, D), jnp.float32),
    )(agg2, hn, w, b, scale)


def _tc_mlp_body(p_ref, c0_ref, c1_ref, w0_ref, b0_ref, w1_ref, b1_ref,
                 w2_ref, b2_ref, out_ref):
    p = p_ref[0, pl.ds(0, G), :] + p_ref[1, pl.ds(0, G), :]
    cnt = c0_ref[pl.ds(0, G), :] + c1_ref[pl.ds(0, G), :]
    cnt = jnp.maximum(cnt, 1.0)
    p = p / cnt
    y = jnp.dot(p, w0_ref[...], preferred_element_type=jnp.float32)
    y = jnp.maximum(y + b0_ref[...], 0.0)
    y = jnp.dot(y, w1_ref[...], preferred_element_type=jnp.float32)
    y = jnp.maximum(y + b1_ref[...], 0.0)
    y = jnp.dot(y, w2_ref[...], preferred_element_type=jnp.float32)
    out_ref[...] = y + b2_ref[...]


def _tc_mlp(pooled2, cnt0, cnt1, w0, b0, w1, b1, w2, b2):
    return pl.pallas_call(
        _tc_mlp_body,
        out_shape=jax.ShapeDtypeStruct((G, 1), jnp.float32),
    )(pooled2, cnt0, cnt1, w0, b0, w1, b1, w2, b2)


def kernel(x, edge_index, ptr, emb, Wc0, bc0, Wc1, bc1, Wc2, bc2, Wc3, bc3,
           Wm0, bm0, Wm1, bm1, Wm2, bm2):
    f32 = jnp.float32
    x_p = jnp.concatenate([x.astype(jnp.int32), jnp.zeros((NP - N,), jnp.int32)])
    trash = jnp.full((EP - E,), NP - 1, jnp.int32)
    src = jnp.concatenate([edge_index[0].astype(jnp.int32), trash])
    dst = jnp.concatenate([edge_index[1].astype(jnp.int32), trash])
    src2d = src.reshape(TILES, NCHUNK, ECHUNK)
    dst2d = dst.reshape(TILES, NCHUNK, ECHUNK)
    srcp = src.reshape(TILES, PNCH, PEC)
    ptr2d = jnp.concatenate(
        [ptr.astype(jnp.int32), jnp.full((NP - N,), G, jnp.int32)]).reshape(
        TILES, ROWS_PER_TILE // 64, 64)
    z2h = jnp.zeros((64, D), f32)
    zh = jnp.zeros((ROWS_PER_SUB, D), f32)
    z1h = jnp.zeros((64,), f32)
    onesh = jnp.ones((PEC,), f32)
    ones_scale = jnp.ones((NP, 1), f32)

    h0, deg0, deg1 = _sc_prep(srcp, x_p, emb, z1h, onesh)
    hn, rdeg = _tc_prep(h0, deg0.reshape(NP, 1), deg1.reshape(NP, 1))
    for i, (w, b) in enumerate(((Wc0, bc0), (Wc1, bc1), (Wc2, bc2), (Wc3, bc3))):
        agg2 = _sc_scatter(hn, src2d, dst2d, zh)
        scale = rdeg if i < 3 else ones_scale
        hn = _tc_layer(agg2, hn, w, b.reshape(1, D), scale)
    pooled2, cnt0, cnt1 = _sc_pool(hn, ptr2d, z2h, z1h, onesh)
    y = _tc_mlp(pooled2, cnt0.reshape(GP, 1), cnt1.reshape(GP, 1),
                Wm0, bm0.reshape(1, D // 2), Wm1, bm1.reshape(1, D // 4),
                Wm2, bm2.reshape(1, 1))
    return y


# speedup vs baseline: 1.0981x; 1.0981x over previous
import functools

import jax
import jax.numpy as jnp
from jax import lax
from jax.experimental import pallas as pl
from jax.experimental.pallas import tpu as pltpu
from jax.experimental.pallas import tpu_sc as plsc

N = 10000
E = 320000
D = 128
G = 512
NP = 10240
GP = 768
NC = 2
NS = 16
TILES = NC * NS
ROWS_PER_TILE = NP // TILES
ROWS_PER_SUB = NP // NS

ECHUNK = 128
NCHUNK = 80
HCHUNK = 16
NBUF = 2
EP = TILES * NCHUNK * ECHUNK

PEC = 128
PNCH = 80

_mesh = plsc.VectorSubcoreMesh(core_axis_name="c", subcore_axis_name="s")


def _wid():
    return lax.axis_index("s") * NC + lax.axis_index("c")


@functools.partial(
    pl.kernel,
    out_type=(
        jax.ShapeDtypeStruct((NP, D), jnp.float32),
        jax.ShapeDtypeStruct((NP,), jnp.float32),
        jax.ShapeDtypeStruct((NP,), jnp.float32),
    ),
    mesh=_mesh,
    scratch_types=[
        pltpu.VMEM((PNCH, PEC), jnp.int32),
        pltpu.VMEM((PEC,), jnp.float32),
        pltpu.VMEM((64,), jnp.int32),
        pltpu.VMEM((64, D), jnp.float32),
        pltpu.VMEM((64,), jnp.float32),
        pltpu.VMEM((ROWS_PER_SUB,), jnp.float32),
        pltpu.VMEM_SHARED((NP,), jnp.float32),
        pltpu.SemaphoreType.DMA,
    ],
)
def _sc_prep(srcp, xp, emb, z1h, onesh, h0_out, deg0_out, deg1_out,
             sidx, onesv, xidx, rows, z1v, dbuf, acc1, sem):
    c = lax.axis_index("c")
    s = lax.axis_index("s")
    wid = _wid()
    pltpu.sync_copy(z1h, z1v)
    pltpu.sync_copy(onesh, onesv)

    def zero_body(j, _):
        pltpu.sync_copy(z1v, acc1.at[pl.ds(s * ROWS_PER_SUB + j * 64, 64)])
        return _
    lax.fori_loop(0, ROWS_PER_SUB // 64, zero_body, None)

    def emb_body(j, _):
        base = wid * ROWS_PER_TILE + j * 64
        pltpu.sync_copy(xp.at[pl.ds(base, 64)], xidx)
        pltpu.async_copy(emb.at[xidx], rows, sem).wait()
        pltpu.sync_copy(rows, h0_out.at[pl.ds(base, 64)])
        return _
    lax.fori_loop(0, ROWS_PER_TILE // 64, emb_body, None)

    pltpu.sync_copy(srcp.at[wid], sidx)
    plsc.subcore_barrier()

    def deg_body(i, _):
        pltpu.sync_copy(onesv, acc1.at[sidx.at[i]], add=True)
        return _
    lax.fori_loop(0, PNCH, deg_body, None)

    plsc.subcore_barrier()
    sl = pl.ds(s * ROWS_PER_SUB, ROWS_PER_SUB)
    pltpu.sync_copy(acc1.at[sl], dbuf)

    @pl.when(c == 0)
    def _w0():
        pltpu.sync_copy(dbuf, deg0_out.at[sl])

    @pl.when(c == 1)
    def _w1():
        pltpu.sync_copy(dbuf, deg1_out.at[sl])


@functools.partial(
    pl.kernel,
    out_type=jax.ShapeDtypeStruct((NC, NP, D), jnp.float32),
    mesh=_mesh,
    scratch_types=[
        pltpu.VMEM((HCHUNK, ECHUNK), jnp.int32),
        pltpu.VMEM((HCHUNK, ECHUNK), jnp.int32),
        pltpu.VMEM((ECHUNK, D), jnp.float32),
        pltpu.VMEM((ECHUNK, D), jnp.float32),
        pltpu.VMEM_SHARED((NP, D), jnp.float32),
        pltpu.SemaphoreType.DMA,
        pltpu.SemaphoreType.DMA,
        pltpu.SemaphoreType.DMA,
        pltpu.SemaphoreType.DMA,
    ],
)
def _sc_scatter(hn, src2d, dst2d, zh, agg_out,
                sidx, didx, r0, r1, acc,
                g0, g1, t0, t1):
    c = lax.axis_index("c")
    s = lax.axis_index("s")
    wid = _wid()
    rows = (r0, r1)
    gsem = (g0, g1)
    tsem = (t0, t1)
    sl = pl.ds(s * ROWS_PER_SUB, ROWS_PER_SUB)

    pltpu.sync_copy(zh, acc.at[sl])
    plsc.subcore_barrier()

    def _g_start(k, b):
        pltpu.async_copy(hn.at[sidx.at[k]], rows[b], gsem[b])

    def _g_wait(k, b):
        pltpu.make_async_copy(hn.at[sidx.at[k]], rows[b], gsem[b]).wait()

    def _t_start(k, b):
        pltpu.async_copy(rows[b], acc.at[didx.at[k]], tsem[b], add=True)

    def _t_wait(k, b):
        pltpu.make_async_copy(rows[b], acc.at[didx.at[k]], tsem[b]).wait()

    for h in range(NCHUNK // HCHUNK):
        pltpu.sync_copy(src2d.at[wid, pl.ds(h * HCHUNK, HCHUNK)], sidx)
        pltpu.sync_copy(dst2d.at[wid, pl.ds(h * HCHUNK, HCHUNK)], didx)
        _g_start(0, 0)

        def group(j, _):
            for b in range(NBUF):
                k = j * NBUF + b
                b1 = 1 - b
                _g_wait(k, b)

                @pl.when(k >= 1)
                def _retire():
                    _t_wait(k - 1, b1)

                @pl.when(k + 1 < HCHUNK)
                def _refill():
                    _g_start(k + 1, b1)
                _t_start(k, b)
            return _
        lax.fori_loop(0, HCHUNK // NBUF, group, None)

        _t_wait(HCHUNK - 1, (HCHUNK - 1) % NBUF)

    plsc.subcore_barrier()
    pltpu.sync_copy(acc.at[sl], agg_out.at[c, sl])


_GROWS = GP // NS


@functools.partial(
    pl.kernel,
    out_type=(
        jax.ShapeDtypeStruct((NC, GP, D), jnp.float32),
        jax.ShapeDtypeStruct((GP,), jnp.float32),
        jax.ShapeDtypeStruct((GP,), jnp.float32),
    ),
    mesh=_mesh,
    scratch_types=[
        pltpu.VMEM((ROWS_PER_TILE // 64, 64), jnp.int32),
        pltpu.VMEM((64, D), jnp.float32),
        pltpu.VMEM((64,), jnp.float32),
        pltpu.VMEM((64, D), jnp.float32),
        pltpu.VMEM((_GROWS,), jnp.float32),
        pltpu.VMEM_SHARED((GP, D), jnp.float32),
        pltpu.VMEM_SHARED((GP,), jnp.float32),
    ],
)
def _sc_pool(h4, ptr2d, z2h, z1h, onesh, pooled_out, cnt0_out, cnt1_out,
             pidx, rowb, onesv, zb, z1v, acc_r, acc_c):
    c = lax.axis_index("c")
    s = lax.axis_index("s")
    wid = _wid()
    pltpu.sync_copy(z2h.at[pl.ds(0, 64)], zb)
    pltpu.sync_copy(z1h.at[pl.ds(0, _GROWS)], z1v)
    pltpu.sync_copy(onesh.at[pl.ds(0, 64)], onesv)
    sl = pl.ds(s * _GROWS, _GROWS)
    pltpu.sync_copy(zb.at[pl.ds(0, _GROWS)], acc_r.at[sl])
    pltpu.sync_copy(z1v, acc_c.at[sl])
    nch = ROWS_PER_TILE // 64
    pltpu.sync_copy(ptr2d.at[wid], pidx)
    plsc.subcore_barrier()

    def body(j, _):
        pltpu.sync_copy(h4.at[pl.ds(wid * ROWS_PER_TILE + j * 64, 64)], rowb)
        pltpu.sync_copy(rowb, acc_r.at[pidx.at[j]], add=True)
        pltpu.sync_copy(onesv, acc_c.at[pidx.at[j]], add=True)
        return _
    lax.fori_loop(0, nch, body, None)

    plsc.subcore_barrier()
    pltpu.sync_copy(acc_r.at[sl], zb.at[pl.ds(0, _GROWS)])
    pltpu.sync_copy(zb.at[pl.ds(0, _GROWS)], pooled_out.at[c, sl])
    pltpu.sync_copy(acc_c.at[sl], z1v)

    @pl.when(c == 0)
    def _w0():
        pltpu.sync_copy(z1v, cnt0_out.at[sl])

    @pl.when(c == 1)
    def _w1():
        pltpu.sync_copy(z1v, cnt1_out.at[sl])


_RB = 512
_NBLK = NP // _RB


def _tc_prep_body(h0_ref, deg0_ref, deg1_ref, hn_ref, rdeg_ref):
    dg = deg0_ref[...] + deg1_ref[...]
    r = lax.rsqrt(dg + 1.0)
    rdeg_ref[...] = r
    hn_ref[...] = h0_ref[...] * r


def _tc_prep(h0, deg0, deg1):
    return pl.pallas_call(
        _tc_prep_body,
        grid=(_NBLK,),
        in_specs=[
            pl.BlockSpec((_RB, D), lambda i: (i, 0)),
            pl.BlockSpec((_RB, 1), lambda i: (i, 0)),
            pl.BlockSpec((_RB, 1), lambda i: (i, 0)),
        ],
        out_specs=[
            pl.BlockSpec((_RB, D), lambda i: (i, 0)),
            pl.BlockSpec((_RB, 1), lambda i: (i, 0)),
        ],
        out_shape=[
            jax.ShapeDtypeStruct((NP, D), jnp.float32),
            jax.ShapeDtypeStruct((NP, 1), jnp.float32),
        ],
    )(h0, deg0, deg1)


def _tc_layer_body(agg_ref, hn_ref, w_ref, b_ref, sc_ref, out_ref):
    a = agg_ref[0] + agg_ref[1] + hn_ref[...]
    y = jnp.dot(a, w_ref[...], preferred_element_type=jnp.float32)
    y = jnp.maximum(y + b_ref[...], 0.0)
    out_ref[...] = y * sc_ref[...]


def _tc_layer(agg2, hn, w, b, scale):
    return pl.pallas_call(
        _tc_layer_body,
        grid=(_NBLK,),
        in_specs=[
            pl.BlockSpec((NC, _RB, D), lambda i: (0, i, 0)),
            pl.BlockSpec((_RB, D), lambda i: (i, 0)),
            pl.BlockSpec((D, D), lambda i: (0, 0)),
            pl.BlockSpec((1, D), lambda i: (0, 0)),
            pl.BlockSpec((_RB, 1), lambda i: (i, 0)),
        ],
        out_specs=pl.BlockSpec((_RB, D), lambda i: (i, 0)),
        out_shape=jax.ShapeDtypeStruct((NP, D), jnp.float32),
    )(agg2, hn, w, b, scale)


def _tc_mlp_body(p_ref, c0_ref, c1_ref, w0_ref, b0_ref, w1_ref, b1_ref,
                 w2_ref, b2_ref, out_ref):
    p = p_ref[0, pl.ds(0, G), :] + p_ref[1, pl.ds(0, G), :]
    cnt = c0_ref[pl.ds(0, G), :] + c1_ref[pl.ds(0, G), :]
    cnt = jnp.maximum(cnt, 1.0)
    p = p / cnt
    y = jnp.dot(p, w0_ref[...], preferred_element_type=jnp.float32)
    y = jnp.maximum(y + b0_ref[...], 0.0)
    y = jnp.dot(y, w1_ref[...], preferred_element_type=jnp.float32)
    y = jnp.maximum(y + b1_ref[...], 0.0)
    y = jnp.dot(y, w2_ref[...], preferred_element_type=jnp.float32)
    out_ref[...] = y + b2_ref[...]


def _tc_mlp(pooled2, cnt0, cnt1, w0, b0, w1, b1, w2, b2):
    return pl.pallas_call(
        _tc_mlp_body,
        out_shape=jax.ShapeDtypeStruct((G, 1), jnp.float32),
    )(pooled2, cnt0, cnt1, w0, b0, w1, b1, w2, b2)


def kernel(x, edge_index, ptr, emb, Wc0, bc0, Wc1, bc1, Wc2, bc2, Wc3, bc3,
           Wm0, bm0, Wm1, bm1, Wm2, bm2):
    f32 = jnp.float32
    x_p = jnp.concatenate([x.astype(jnp.int32), jnp.zeros((NP - N,), jnp.int32)])
    trash = jnp.full((EP - E,), NP - 1, jnp.int32)
    src = jnp.concatenate([edge_index[0].astype(jnp.int32), trash])
    dst = jnp.concatenate([edge_index[1].astype(jnp.int32), trash])
    src2d = src.reshape(TILES, NCHUNK, ECHUNK)
    dst2d = dst.reshape(TILES, NCHUNK, ECHUNK)
    srcp = src.reshape(TILES, PNCH, PEC)
    ptr2d = jnp.concatenate(
        [ptr.astype(jnp.int32), jnp.full((NP - N,), G, jnp.int32)]).reshape(
        TILES, ROWS_PER_TILE // 64, 64)
    z2h = jnp.zeros((64, D), f32)
    zh = jnp.zeros((ROWS_PER_SUB, D), f32)
    z1h = jnp.zeros((64,), f32)
    onesh = jnp.ones((PEC,), f32)
    ones_scale = jnp.ones((NP, 1), f32)

    h0, deg0, deg1 = _sc_prep(srcp, x_p, emb, z1h, onesh)
    hn, rdeg = _tc_prep(h0, deg0.reshape(NP, 1), deg1.reshape(NP, 1))
    for i, (w, b) in enumerate(((Wc0, bc0), (Wc1, bc1), (Wc2, bc2), (Wc3, bc3))):
        agg2 = _sc_scatter(hn, src2d, dst2d, zh)
        scale = rdeg if i < 3 else ones_scale
        hn = _tc_layer(agg2, hn, w, b.reshape(1, D), scale)
    pooled2, cnt0, cnt1 = _sc_pool(hn, ptr2d, z2h, z1h, onesh)
    y = _tc_mlp(pooled2, cnt0.reshape(GP, 1), cnt1.reshape(GP, 1),
                Wm0, bm0.reshape(1, D // 2), Wm1, bm1.reshape(1, D // 4),
                Wm2, bm2.reshape(1, 1))
    return y


# speedup vs baseline: 1.1497x; 1.0470x over previous
import functools

import jax
import jax.numpy as jnp
from jax import lax
from jax.experimental import pallas as pl
from jax.experimental.pallas import tpu as pltpu
from jax.experimental.pallas import tpu_sc as plsc

N = 10000
E = 320000
D = 128
G = 512
NP = 10240
GP = 768
NC = 2
NS = 16
TILES = NC * NS
ROWS_PER_TILE = NP // TILES
ROWS_PER_SUB = NP // NS

ECHUNK = 128
NCHUNK = 80
HCHUNK = 40
NBUF = 2
EP = TILES * NCHUNK * ECHUNK

PEC = 128
PNCH = 80

_mesh = plsc.VectorSubcoreMesh(core_axis_name="c", subcore_axis_name="s")


def _wid():
    return lax.axis_index("s") * NC + lax.axis_index("c")


@functools.partial(
    pl.kernel,
    out_type=(
        jax.ShapeDtypeStruct((NP, D), jnp.float32),
        jax.ShapeDtypeStruct((NP,), jnp.float32),
        jax.ShapeDtypeStruct((NP,), jnp.float32),
    ),
    mesh=_mesh,
    scratch_types=[
        pltpu.VMEM((PNCH, PEC), jnp.int32),
        pltpu.VMEM((PEC,), jnp.float32),
        pltpu.VMEM((64,), jnp.int32),
        pltpu.VMEM((64, D), jnp.float32),
        pltpu.VMEM((64,), jnp.float32),
        pltpu.VMEM((ROWS_PER_SUB,), jnp.float32),
        pltpu.VMEM_SHARED((NP,), jnp.float32),
        pltpu.SemaphoreType.DMA,
    ],
)
def _sc_prep(srcp, xp, emb, z1h, onesh, h0_out, deg0_out, deg1_out,
             sidx, onesv, xidx, rows, z1v, dbuf, acc1, sem):
    c = lax.axis_index("c")
    s = lax.axis_index("s")
    wid = _wid()
    pltpu.sync_copy(z1h, z1v)
    pltpu.sync_copy(onesh, onesv)

    def zero_body(j, _):
        pltpu.sync_copy(z1v, acc1.at[pl.ds(s * ROWS_PER_SUB + j * 64, 64)])
        return _
    lax.fori_loop(0, ROWS_PER_SUB // 64, zero_body, None)

    def emb_body(j, _):
        base = wid * ROWS_PER_TILE + j * 64
        pltpu.sync_copy(xp.at[pl.ds(base, 64)], xidx)
        pltpu.async_copy(emb.at[xidx], rows, sem).wait()
        pltpu.sync_copy(rows, h0_out.at[pl.ds(base, 64)])
        return _
    lax.fori_loop(0, ROWS_PER_TILE // 64, emb_body, None)

    pltpu.sync_copy(srcp.at[wid], sidx)
    plsc.subcore_barrier()

    def deg_body(i, _):
        pltpu.sync_copy(onesv, acc1.at[sidx.at[i]], add=True)
        return _
    lax.fori_loop(0, PNCH, deg_body, None)

    plsc.subcore_barrier()
    sl = pl.ds(s * ROWS_PER_SUB, ROWS_PER_SUB)
    pltpu.sync_copy(acc1.at[sl], dbuf)

    @pl.when(c == 0)
    def _w0():
        pltpu.sync_copy(dbuf, deg0_out.at[sl])

    @pl.when(c == 1)
    def _w1():
        pltpu.sync_copy(dbuf, deg1_out.at[sl])


@functools.partial(
    pl.kernel,
    out_type=jax.ShapeDtypeStruct((NC, NP, D), jnp.float32),
    mesh=_mesh,
    scratch_types=[
        pltpu.VMEM((HCHUNK, ECHUNK), jnp.int32),
        pltpu.VMEM((HCHUNK, ECHUNK), jnp.int32),
        pltpu.VMEM((ECHUNK, D), jnp.float32),
        pltpu.VMEM((ECHUNK, D), jnp.float32),
        pltpu.VMEM_SHARED((NP, D), jnp.float32),
        pltpu.SemaphoreType.DMA,
        pltpu.SemaphoreType.DMA,
    ],
)
def _sc_scatter(hn, src2d, dst2d, zh, agg_out,
                sidx, didx, r0, r1, acc, g0, g1):
    c = lax.axis_index("c")
    s = lax.axis_index("s")
    wid = _wid()
    rows = (r0, r1)
    gsem = (g0, g1)
    sl = pl.ds(s * ROWS_PER_SUB, ROWS_PER_SUB)

    zr = ROWS_PER_SUB // 2
    pltpu.sync_copy(zh, acc.at[pl.ds(s * ROWS_PER_SUB, zr)])
    pltpu.sync_copy(zh, acc.at[pl.ds(s * ROWS_PER_SUB + zr, zr)])
    plsc.subcore_barrier()

    def _g_start(k, b):
        pltpu.async_copy(hn.at[sidx.at[k]], rows[b], gsem[b])

    def _g_wait(k, b):
        pltpu.make_async_copy(hn.at[sidx.at[k]], rows[b], gsem[b]).wait()

    for h in range(NCHUNK // HCHUNK):
        pltpu.sync_copy(src2d.at[wid, pl.ds(h * HCHUNK, HCHUNK)], sidx)
        pltpu.sync_copy(dst2d.at[wid, pl.ds(h * HCHUNK, HCHUNK)], didx)
        _g_start(0, 0)

        def pair(kk, _):
            e0 = 2 * kk
            _g_start(e0 + 1, 1)
            _g_wait(e0, 0)
            pltpu.sync_copy(rows[0], acc.at[didx.at[e0]], add=True)

            @pl.when(e0 + 2 < HCHUNK)
            def _prefetch():
                _g_start(e0 + 2, 0)

            _g_wait(e0 + 1, 1)
            pltpu.sync_copy(rows[1], acc.at[didx.at[e0 + 1]], add=True)
            return _
        lax.fori_loop(0, HCHUNK // 2, pair, None)

    plsc.subcore_barrier()
    pltpu.sync_copy(acc.at[sl], agg_out.at[c, sl])


_GROWS = GP // NS


@functools.partial(
    pl.kernel,
    out_type=(
        jax.ShapeDtypeStruct((NC, GP, D), jnp.float32),
        jax.ShapeDtypeStruct((GP,), jnp.float32),
        jax.ShapeDtypeStruct((GP,), jnp.float32),
    ),
    mesh=_mesh,
    scratch_types=[
        pltpu.VMEM((ROWS_PER_TILE // 64, 64), jnp.int32),
        pltpu.VMEM((64, D), jnp.float32),
        pltpu.VMEM((64,), jnp.float32),
        pltpu.VMEM((64, D), jnp.float32),
        pltpu.VMEM((_GROWS,), jnp.float32),
        pltpu.VMEM_SHARED((GP, D), jnp.float32),
        pltpu.VMEM_SHARED((GP,), jnp.float32),
    ],
)
def _sc_pool(h4, ptr2d, z2h, z1h, onesh, pooled_out, cnt0_out, cnt1_out,
             pidx, rowb, onesv, zb, z1v, acc_r, acc_c):
    c = lax.axis_index("c")
    s = lax.axis_index("s")
    wid = _wid()
    pltpu.sync_copy(z2h.at[pl.ds(0, 64)], zb)
    pltpu.sync_copy(z1h.at[pl.ds(0, _GROWS)], z1v)
    pltpu.sync_copy(onesh.at[pl.ds(0, 64)], onesv)
    sl = pl.ds(s * _GROWS, _GROWS)
    pltpu.sync_copy(zb.at[pl.ds(0, _GROWS)], acc_r.at[sl])
    pltpu.sync_copy(z1v, acc_c.at[sl])
    nch = ROWS_PER_TILE // 64
    pltpu.sync_copy(ptr2d.at[wid], pidx)
    plsc.subcore_barrier()

    def body(j, _):
        pltpu.sync_copy(h4.at[pl.ds(wid * ROWS_PER_TILE + j * 64, 64)], rowb)
        pltpu.sync_copy(rowb, acc_r.at[pidx.at[j]], add=True)
        pltpu.sync_copy(onesv, acc_c.at[pidx.at[j]], add=True)
        return _
    lax.fori_loop(0, nch, body, None)

    plsc.subcore_barrier()
    pltpu.sync_copy(acc_r.at[sl], zb.at[pl.ds(0, _GROWS)])
    pltpu.sync_copy(zb.at[pl.ds(0, _GROWS)], pooled_out.at[c, sl])
    pltpu.sync_copy(acc_c.at[sl], z1v)

    @pl.when(c == 0)
    def _w0():
        pltpu.sync_copy(z1v, cnt0_out.at[sl])

    @pl.when(c == 1)
    def _w1():
        pltpu.sync_copy(z1v, cnt1_out.at[sl])


_RB = 512
_NBLK = NP // _RB


def _tc_prep_body(h0_ref, deg0_ref, deg1_ref, hn_ref, rdeg_ref):
    dg = deg0_ref[...] + deg1_ref[...]
    r = lax.rsqrt(dg + 1.0)
    rdeg_ref[...] = r
    hn_ref[...] = h0_ref[...] * r


def _tc_prep(h0, deg0, deg1):
    return pl.pallas_call(
        _tc_prep_body,
        grid=(_NBLK,),
        in_specs=[
            pl.BlockSpec((_RB, D), lambda i: (i, 0)),
            pl.BlockSpec((_RB, 1), lambda i: (i, 0)),
            pl.BlockSpec((_RB, 1), lambda i: (i, 0)),
        ],
        out_specs=[
            pl.BlockSpec((_RB, D), lambda i: (i, 0)),
            pl.BlockSpec((_RB, 1), lambda i: (i, 0)),
        ],
        out_shape=[
            jax.ShapeDtypeStruct((NP, D), jnp.float32),
            jax.ShapeDtypeStruct((NP, 1), jnp.float32),
        ],
    )(h0, deg0, deg1)


def _tc_layer_body(agg_ref, hn_ref, w_ref, b_ref, sc_ref, out_ref):
    a = agg_ref[0] + agg_ref[1] + hn_ref[...]
    y = jnp.dot(a, w_ref[...], preferred_element_type=jnp.float32)
    y = jnp.maximum(y + b_ref[...], 0.0)
    out_ref[...] = y * sc_ref[...]


def _tc_layer(agg2, hn, w, b, scale):
    return pl.pallas_call(
        _tc_layer_body,
        grid=(_NBLK,),
        in_specs=[
            pl.BlockSpec((NC, _RB, D), lambda i: (0, i, 0)),
            pl.BlockSpec((_RB, D), lambda i: (i, 0)),
            pl.BlockSpec((D, D), lambda i: (0, 0)),
            pl.BlockSpec((1, D), lambda i: (0, 0)),
            pl.BlockSpec((_RB, 1), lambda i: (i, 0)),
        ],
        out_specs=pl.BlockSpec((_RB, D), lambda i: (i, 0)),
        out_shape=jax.ShapeDtypeStruct((NP, D), jnp.float32),
    )(agg2, hn, w, b, scale)


def _tc_mlp_body(p_ref, c0_ref, c1_ref, w0_ref, b0_ref, w1_ref, b1_ref,
                 w2_ref, b2_ref, out_ref):
    p = p_ref[0, pl.ds(0, G), :] + p_ref[1, pl.ds(0, G), :]
    cnt = c0_ref[pl.ds(0, G), :] + c1_ref[pl.ds(0, G), :]
    cnt = jnp.maximum(cnt, 1.0)
    p = p / cnt
    y = jnp.dot(p, w0_ref[...], preferred_element_type=jnp.float32)
    y = jnp.maximum(y + b0_ref[...], 0.0)
    y = jnp.dot(y, w1_ref[...], preferred_element_type=jnp.float32)
    y = jnp.maximum(y + b1_ref[...], 0.0)
    y = jnp.dot(y, w2_ref[...], preferred_element_type=jnp.float32)
    out_ref[...] = y + b2_ref[...]


def _tc_mlp(pooled2, cnt0, cnt1, w0, b0, w1, b1, w2, b2):
    return pl.pallas_call(
        _tc_mlp_body,
        out_shape=jax.ShapeDtypeStruct((G, 1), jnp.float32),
    )(pooled2, cnt0, cnt1, w0, b0, w1, b1, w2, b2)


def kernel(x, edge_index, ptr, emb, Wc0, bc0, Wc1, bc1, Wc2, bc2, Wc3, bc3,
           Wm0, bm0, Wm1, bm1, Wm2, bm2):
    f32 = jnp.float32
    x_p = jnp.concatenate([x.astype(jnp.int32), jnp.zeros((NP - N,), jnp.int32)])
    trash = jnp.full((EP - E,), NP - 1, jnp.int32)
    src = jnp.concatenate([edge_index[0].astype(jnp.int32), trash])
    dst = jnp.concatenate([edge_index[1].astype(jnp.int32), trash])
    src2d = src.reshape(TILES, NCHUNK, ECHUNK)
    dst2d = dst.reshape(TILES, NCHUNK, ECHUNK)
    srcp = src.reshape(TILES, PNCH, PEC)
    ptr2d = jnp.concatenate(
        [ptr.astype(jnp.int32), jnp.full((NP - N,), G, jnp.int32)]).reshape(
        TILES, ROWS_PER_TILE // 64, 64)
    z2h = jnp.zeros((64, D), f32)
    zh = jnp.zeros((ROWS_PER_SUB // 2, D), f32)
    z1h = jnp.zeros((64,), f32)
    onesh = jnp.ones((PEC,), f32)
    ones_scale = jnp.ones((NP, 1), f32)

    h0, deg0, deg1 = _sc_prep(srcp, x_p, emb, z1h, onesh)
    hn, rdeg = _tc_prep(h0, deg0.reshape(NP, 1), deg1.reshape(NP, 1))
    for i, (w, b) in enumerate(((Wc0, bc0), (Wc1, bc1), (Wc2, bc2), (Wc3, bc3))):
        agg2 = _sc_scatter(hn, src2d, dst2d, zh)
        scale = rdeg if i < 3 else ones_scale
        hn = _tc_layer(agg2, hn, w, b.reshape(1, D), scale)
    pooled2, cnt0, cnt1 = _sc_pool(hn, ptr2d, z2h, z1h, onesh)
    y = _tc_mlp(pooled2, cnt0.reshape(GP, 1), cnt1.reshape(GP, 1),
                Wm0, bm0.reshape(1, D // 2), Wm1, bm1.reshape(1, D // 4),
                Wm2, bm2.reshape(1, 1))
    return y
